# Initial kernel scaffold; baseline (speedup 1.0000x reference)
import jax
import jax.numpy as jnp
from jax.experimental import pallas as pl


def kernel(aspect_mask, position_embeddings):
    raise NotImplementedError("write your pallas kernel here")


# speedup vs baseline: 4.4387x; 4.4387x over previous
import functools

import jax
import jax.numpy as jnp
from jax import lax
from jax.experimental import pallas as pl
from jax.experimental.pallas import tpu as pltpu
from jax.experimental.pallas import tpu_sc as plsc

MAX_POSITION = 50
EMBED_DIM = 64
NUM_EMB = 2 * MAX_POSITION + 1
BATCH = 4096
SEQ = 200
EXT_ROWS = 2 * SEQ - 1
ROW_F32 = SEQ * EMBED_DIM

NUM_CORES = 2
NUM_SUBCORES = 16
NUM_WORKERS = NUM_CORES * NUM_SUBCORES
ROWS_PER_WORKER = BATCH // NUM_WORKERS
LANES = 16
GROUPS = ROWS_PER_WORKER // LANES


def _sc_body(maskt_hbm, table_hbm, out_hbm, mask_v, ext_v, sem):
    wid = lax.axis_index("s") * NUM_CORES + lax.axis_index("c")
    base = wid * ROWS_PER_WORKER

    pltpu.sync_copy(maskt_hbm.at[:, pl.ds(base, ROWS_PER_WORKER)], mask_v)

    mid = (SEQ - 1 - MAX_POSITION) * EMBED_DIM
    pltpu.sync_copy(table_hbm, ext_v.at[pl.ds(mid, NUM_EMB * EMBED_DIM)])

    head_src = [ext_v[pl.ds(mid + k * LANES, LANES)] for k in range(4)]
    tail_off = (SEQ - 1 + MAX_POSITION) * EMBED_DIM
    tail_src = [ext_v[pl.ds(tail_off + k * LANES, LANES)] for k in range(4)]

    def fill(i, _):
        off_h = i * EMBED_DIM
        off_t = (SEQ + MAX_POSITION) * EMBED_DIM + i * EMBED_DIM
        for k in range(4):
            ext_v[pl.ds(off_h + k * LANES, LANES)] = head_src[k]
            ext_v[pl.ds(off_t + k * LANES, LANES)] = tail_src[k]
        return 0

    lax.fori_loop(0, SEQ - 1 - MAX_POSITION, fill, 0)

    def group(g, _):
        col = g * LANES

        def accum(s, carry):
            acc_s, acc_c = carry
            m = mask_v[s, pl.ds(col, LANES)]
            return acc_s + m * s.astype(jnp.float32), acc_c + m

        acc_s, acc_c = lax.fori_loop(
            0,
            SEQ,
            accum,
            (jnp.zeros((LANES,), jnp.float32), jnp.zeros((LANES,), jnp.float32)),
        )
        d = jnp.maximum(acc_c, 1.0)
        q = (acc_s / d).astype(jnp.int32)
        r = acc_s - q.astype(jnp.float32) * d
        q = jnp.where(r >= d, q + 1, q)
        q = jnp.where(r < 0.0, q - 1, q)
        starts = ((SEQ - 1) - q) * EMBED_DIM

        copies = []
        for j in range(LANES):
            copies.append(
                pltpu.async_copy(
                    ext_v.at[pl.ds(pl.multiple_of(starts[j], EMBED_DIM), ROW_F32)],
                    out_hbm.at[pl.ds((base + col + j) * ROW_F32, ROW_F32)],
                    sem,
                )
            )
        for cp in copies:
            cp.wait()
        return 0

    lax.fori_loop(0, GROUPS, group, 0)


@jax.jit
def _run(maskt, table_flat):
    mesh = plsc.VectorSubcoreMesh(core_axis_name="c", subcore_axis_name="s")
    f = functools.partial(
        pl.kernel,
        mesh=mesh,
        out_type=jax.ShapeDtypeStruct((BATCH * ROW_F32,), jnp.float32),
        scratch_types=[
            pltpu.VMEM((SEQ, ROWS_PER_WORKER), jnp.float32),
            pltpu.VMEM((EXT_ROWS * EMBED_DIM,), jnp.float32),
            pltpu.SemaphoreType.DMA,
        ],
    )(_sc_body)
    return f(maskt, table_flat)


def kernel(aspect_mask, position_embeddings):
    maskt = aspect_mask.astype(jnp.float32).T
    table_flat = position_embeddings.reshape(-1)
    out = _run(maskt, table_flat)
    return out.reshape(BATCH, SEQ, EMBED_DIM)


# speedup vs baseline: 5.0873x; 1.1461x over previous
import functools

import jax
import jax.numpy as jnp
from jax import lax
from jax.experimental import pallas as pl
from jax.experimental.pallas import tpu as pltpu
from jax.experimental.pallas import tpu_sc as plsc

MAX_POSITION = 50
EMBED_DIM = 64
NUM_EMB = 2 * MAX_POSITION + 1
BATCH = 4096
SEQ = 200
EXT_ROWS = 2 * SEQ - 1
ROW_F32 = SEQ * EMBED_DIM

NUM_CORES = 2
NUM_SUBCORES = 16
NUM_WORKERS = NUM_CORES * NUM_SUBCORES
ROWS_PER_WORKER = BATCH // NUM_WORKERS
LANES = 16
GROUPS = ROWS_PER_WORKER // LANES


def _sc_body(maskf_hbm, table_hbm, out_hbm, mask_v, ext_v, sem):
    wid = lax.axis_index("s") * NUM_CORES + lax.axis_index("c")
    base = wid * ROWS_PER_WORKER

    pltpu.sync_copy(
        maskf_hbm.at[pl.ds(base * SEQ, ROWS_PER_WORKER * SEQ)], mask_v
    )

    mid = (SEQ - 1 - MAX_POSITION) * EMBED_DIM
    pltpu.sync_copy(table_hbm, ext_v.at[pl.ds(mid, NUM_EMB * EMBED_DIM)])

    head_src = [ext_v[pl.ds(mid + k * LANES, LANES)] for k in range(4)]
    tail_off = (SEQ - 1 + MAX_POSITION) * EMBED_DIM
    tail_src = [ext_v[pl.ds(tail_off + k * LANES, LANES)] for k in range(4)]

    def fill(i, _):
        off_h = i * EMBED_DIM
        off_t = (SEQ + MAX_POSITION) * EMBED_DIM + i * EMBED_DIM
        for k in range(4):
            ext_v[pl.ds(off_h + k * LANES, LANES)] = head_src[k]
            ext_v[pl.ds(off_t + k * LANES, LANES)] = tail_src[k]
        return 0

    lax.fori_loop(0, SEQ - 1 - MAX_POSITION, fill, 0)

    def group(g, _):
        col = g * LANES
        row_base = (col + lax.iota(jnp.int32, LANES)) * SEQ

        def accum(s, carry):
            acc_s, acc_c = carry
            m = plsc.load_gather(mask_v, [row_base + s])
            return acc_s + m * s.astype(jnp.float32), acc_c + m

        acc_s, acc_c = lax.fori_loop(
            0,
            SEQ,
            accum,
            (jnp.zeros((LANES,), jnp.float32), jnp.zeros((LANES,), jnp.float32)),
        )
        d = jnp.maximum(acc_c, 1.0)
        q = (acc_s / d).astype(jnp.int32)
        r = acc_s - q.astype(jnp.float32) * d
        q = jnp.where(r >= d, q + 1, q)
        q = jnp.where(r < 0.0, q - 1, q)
        starts = ((SEQ - 1) - q) * EMBED_DIM

        copies = []
        for j in range(LANES):
            copies.append(
                pltpu.async_copy(
                    ext_v.at[pl.ds(pl.multiple_of(starts[j], EMBED_DIM), ROW_F32)],
                    out_hbm.at[pl.ds((base + col + j) * ROW_F32, ROW_F32)],
                    sem,
                )
            )
        for cp in copies:
            cp.wait()
        return 0

    lax.fori_loop(0, GROUPS, group, 0)


@jax.jit
def _run(maskf, table_flat):
    mesh = plsc.VectorSubcoreMesh(core_axis_name="c", subcore_axis_name="s")
    f = functools.partial(
        pl.kernel,
        mesh=mesh,
        compiler_params=pltpu.CompilerParams(needs_layout_passes=False),
        out_type=jax.ShapeDtypeStruct((BATCH * ROW_F32,), jnp.float32),
        scratch_types=[
            pltpu.VMEM((ROWS_PER_WORKER * SEQ,), jnp.float32),
            pltpu.VMEM((EXT_ROWS * EMBED_DIM,), jnp.float32),
            pltpu.SemaphoreType.DMA,
        ],
    )(_sc_body)
    return f(maskf, table_flat)


def kernel(aspect_mask, position_embeddings):
    maskf = aspect_mask.astype(jnp.float32).reshape(-1)
    table_flat = position_embeddings.reshape(-1)
    out = _run(maskf, table_flat)
    return out.reshape(BATCH, SEQ, EMBED_DIM)
